# baseline (speedup 1.0000x reference)
import jax
import jax.numpy as jnp
from jax.experimental import pallas as pl

_EMB = 64
_POS = 588
_DIN = 16
_B = 1024
_NCH = 8

_BB = 16


def _kernel_body(cb_ref, emb_ref, w_ref, b_ref, x_ref, o_ref):
    cb = cb_ref[0, :]
    iota = jax.lax.broadcasted_iota(jnp.int32, (_POS, _NCH), 1)
    onehot = (cb[:, None] == iota).astype(jnp.float32)
    y = jnp.dot(onehot, emb_ref[...], preferred_element_type=jnp.float32)
    y = y + b_ref[0, :]
    xb = x_ref[...]
    d = jax.lax.dot_general(
        xb, w_ref[...], (((2,), (0,)), ((), ())),
        preferred_element_type=jnp.float32)
    o_ref[...] = d + y[None, :, :]


def kernel(x, emb_table, W, b, channel_base):
    cb = channel_base.astype(jnp.int32).reshape(1, _POS)
    b2 = b.reshape(1, _EMB)
    grid = (_B // _BB,)
    return pl.pallas_call(
        _kernel_body,
        grid=grid,
        in_specs=[
            pl.BlockSpec((1, _POS), lambda i: (0, 0)),
            pl.BlockSpec((_NCH, _EMB), lambda i: (0, 0)),
            pl.BlockSpec((_DIN, _EMB), lambda i: (0, 0)),
            pl.BlockSpec((1, _EMB), lambda i: (0, 0)),
            pl.BlockSpec((_BB, _POS, _DIN), lambda i: (i, 0, 0)),
        ],
        out_specs=pl.BlockSpec((_BB, _POS, _EMB), lambda i: (i, 0, 0)),
        out_shape=jax.ShapeDtypeStruct((_B, _POS, _EMB), jnp.float32),
    )(cb, emb_table, W, b2, x)


# speedup vs baseline: 1.5373x; 1.5373x over previous
import jax
import jax.numpy as jnp
from jax.experimental import pallas as pl

_EMB = 64
_POS = 588
_DIN = 16
_B = 1024
_NCH = 8

_G = 4
_PG = _POS // _G
_KP = _G * _DIN
_NP = _G * _EMB

_BB = 16


def _kernel_body(cb_ref, emb_ref, w_ref, b_ref, x_ref, o_ref):
    iota = jax.lax.broadcasted_iota(jnp.int32, (_PG, _NCH), 1)
    oh = jnp.concatenate(
        [(cb_ref[:, g][:, None] == iota).astype(jnp.float32) for g in range(_G)],
        axis=1)
    y = jnp.dot(oh, emb_ref[...], preferred_element_type=jnp.float32)
    y = y + b_ref[0, :]
    d = jax.lax.dot_general(
        x_ref[...], w_ref[...], (((2,), (0,)), ((), ())),
        preferred_element_type=jnp.float32)
    o_ref[...] = d + y[None, :, :]


def kernel(x, emb_table, W, b, channel_base):
    xg = x.reshape(_B, _PG, _KP)
    eye = jnp.eye(_G, dtype=jnp.float32)
    Wg = jnp.kron(eye, W)
    embg = jnp.kron(eye, emb_table)
    bg = jnp.tile(b, _G).reshape(1, _NP)
    cb = channel_base.astype(jnp.int32).reshape(_PG, _G)
    grid = (_B // _BB,)
    out = pl.pallas_call(
        _kernel_body,
        grid=grid,
        in_specs=[
            pl.BlockSpec((_PG, _G), lambda i: (0, 0)),
            pl.BlockSpec((_G * _NCH, _NP), lambda i: (0, 0)),
            pl.BlockSpec((_KP, _NP), lambda i: (0, 0)),
            pl.BlockSpec((1, _NP), lambda i: (0, 0)),
            pl.BlockSpec((_BB, _PG, _KP), lambda i: (i, 0, 0)),
        ],
        out_specs=pl.BlockSpec((_BB, _PG, _NP), lambda i: (i, 0, 0)),
        out_shape=jax.ShapeDtypeStruct((_B, _PG, _NP), jnp.float32),
    )(cb, embg, Wg, bg, xg)
    return out.reshape(_B, _POS, _EMB)
